# baseline (speedup 1.0000x reference)
import functools

import jax
import jax.numpy as jnp
from jax import lax
from jax.experimental import pallas as pl
from jax.experimental.pallas import tpu as pltpu
from jax.experimental.pallas import tpu_sc as plsc

N_TOK = 2048
N_EMB = 1024
N_EXP = 64
D_EXP = 64
TILE = 64
NPAD = 2816


def _route_kernel(x_ref, wr_ref, pos_ref, off_ref, inv_ref):
    xf = x_ref[...]
    wr = wr_ref[...]
    lg = lax.dot_general(
        wr, xf, (((1,), (1,)), ((), ())),
        preferred_element_type=jnp.float32)
    rowi = lax.broadcasted_iota(jnp.int32, (N_EXP, N_TOK), 0)
    maxv = jnp.max(lg, axis=0, keepdims=True)
    eid = jnp.min(jnp.where(lg == maxv, rowi, N_EXP), axis=0, keepdims=True)
    onehot = (rowi == eid).astype(jnp.float32)
    c = onehot
    d = 1
    while d < N_TOK:
        c = c + jnp.concatenate(
            [jnp.zeros((N_EXP, d), jnp.float32), c[:, : N_TOK - d]], axis=1)
        d *= 2
    rank = jnp.sum(c * onehot, axis=0, keepdims=True) - 1.0
    counts = c[:, N_TOK - 1 : N_TOK]
    pcnt = jnp.ceil(counts / 8.0) * 8.0
    ri = lax.broadcasted_iota(jnp.int32, (N_EXP, N_EXP), 0)
    ci = lax.broadcasted_iota(jnp.int32, (N_EXP, N_EXP), 1)
    tril = (ci < ri).astype(jnp.float32)
    pstart = lax.dot_general(
        tril, pcnt, (((1,), (0,)), ((), ())),
        preferred_element_type=jnp.float32)
    startn = jnp.sum(onehot * pstart, axis=0, keepdims=True)
    posr = startn + rank
    pos_ref[...] = posr.astype(jnp.int32)
    off_ref[...] = jnp.concatenate([pstart, counts], axis=1).astype(jnp.int32)
    piota = lax.broadcasted_iota(jnp.int32, (NPAD, 1), 0).astype(jnp.float32)
    m = (piota == posr).astype(jnp.float32)
    niota = lax.broadcasted_iota(
        jnp.int32, (N_TOK, 1), 0).astype(jnp.float32)
    inv = lax.dot_general(m, niota, (((1,), (0,)), ((), ())),
                          preferred_element_type=jnp.float32)
    inv_ref[...] = inv.astype(jnp.int32)


def _route(x_flat, wr):
    pos2d, offcnt, inv = pl.pallas_call(
        _route_kernel,
        out_shape=(
            jax.ShapeDtypeStruct((1, N_TOK), jnp.int32),
            jax.ShapeDtypeStruct((N_EXP, 2), jnp.int32),
            jax.ShapeDtypeStruct((NPAD, 1), jnp.int32),
        ),
    )(x_flat, wr)
    return pos2d.reshape(N_TOK), offcnt.reshape(2 * N_EXP), inv.reshape(NPAD)


def _sc_mesh():
    info = plsc.get_sparse_core_info()
    return (
        plsc.VectorSubcoreMesh(core_axis_name="c", subcore_axis_name="s"),
        info.num_cores,
        info.num_cores * info.num_subcores,
    )


def _sc_gather(table, idx):
    mesh, nc, nw = _sc_mesh()
    n_out = idx.shape[0]
    bpw = n_out // nw

    @functools.partial(
        pl.kernel,
        out_type=jax.ShapeDtypeStruct((n_out, N_EMB), jnp.float32),
        mesh=mesh,
        scratch_types=[
            pltpu.VMEM((bpw,), jnp.int32),
            pltpu.VMEM((bpw, N_EMB), jnp.float32),
            pltpu.SemaphoreType.DMA,
        ],
    )
    def k(table_hbm, idx_hbm, out_hbm, idx_v, rows_v, sem):
        wid = lax.axis_index("s") * nc + lax.axis_index("c")
        base = wid * bpw
        pltpu.sync_copy(idx_hbm.at[pl.ds(base, bpw)], idx_v)
        pltpu.async_copy(table_hbm.at[idx_v], rows_v, sem).wait()
        pltpu.sync_copy(rows_v, out_hbm.at[pl.ds(base, bpw)])

    return k(table, idx)


GEXP = 8


def _mm_kernel(off_ref, xs_ref, w1a_ref, w1b_ref, w2_ref, out_ref):
    g = pl.program_id(0)

    for j in range(GEXP):
        e = g * GEXP + j
        start = off_ref[2 * e]
        cnt = off_ref[2 * e + 1]
        w1a = w1a_ref[j]
        w1b = w1b_ref[j]
        w2 = w2_ref[j]
        nt = (cnt + TILE - 1) // TILE

        def body(t, carry, w1a=w1a, w1b=w1b, w2=w2, start=start):
            base = pl.multiple_of(start + t * TILE, 8)
            xt = xs_ref[pl.ds(base, TILE), :]
            h = (jnp.dot(xt[:, : N_EMB // 2], w1a,
                         preferred_element_type=jnp.float32)
                 + jnp.dot(xt[:, N_EMB // 2 :], w1b,
                           preferred_element_type=jnp.float32))
            h = jnp.maximum(h, 0.0)
            out_ref[pl.ds(base, TILE), :] = jnp.dot(
                h * h, w2, preferred_element_type=jnp.float32)
            return carry

        body(0, 0)
        lax.fori_loop(1, nt, body, 0)


def _grouped_mm(x_sorted, w1, w2, offcnt):
    grid_spec = pltpu.PrefetchScalarGridSpec(
        num_scalar_prefetch=1,
        grid=(N_EXP // GEXP,),
        in_specs=[
            pl.BlockSpec((NPAD, N_EMB), lambda e, off: (0, 0)),
            pl.BlockSpec((GEXP, N_EMB // 2, D_EXP), lambda e, off: (e, 0, 0)),
            pl.BlockSpec((GEXP, N_EMB // 2, D_EXP), lambda e, off: (e, 1, 0)),
            pl.BlockSpec((GEXP, D_EXP, N_EMB), lambda e, off: (e, 0, 0)),
        ],
        out_specs=pl.BlockSpec((NPAD, N_EMB), lambda e, off: (0, 0)),
    )
    return pl.pallas_call(
        _mm_kernel,
        grid_spec=grid_spec,
        out_shape=jax.ShapeDtypeStruct((NPAD, N_EMB), jnp.float32),
    )(offcnt, x_sorted, w1, w1, w2)


def kernel(x, Wr, W1, W2):
    b, t, c = x.shape
    x_flat = x.reshape(b * t, c)
    pos, offcnt, inv = _route(x_flat, Wr)
    x_sorted = _sc_gather(x_flat, inv)
    out_sorted = _grouped_mm(x_sorted, W1, W2, offcnt)
    out_flat = _sc_gather(out_sorted, pos)
    return out_flat.reshape(b, t, c)


# speedup vs baseline: 1.5325x; 1.5325x over previous
import functools

import jax
import jax.numpy as jnp
from jax import lax
from jax.experimental import pallas as pl
from jax.experimental.pallas import tpu as pltpu
from jax.experimental.pallas import tpu_sc as plsc

N_TOK = 2048
N_EMB = 1024
N_EXP = 64
D_EXP = 64
TILE = 64
NPAD = N_TOK + N_EXP * 8 + TILE


def _route_kernel(x_ref, wr_ref, pos_ref, off_ref):
    xf = x_ref[...]
    wr = wr_ref[...]
    lg = lax.dot_general(
        wr, xf, (((1,), (1,)), ((), ())),
        preferred_element_type=jnp.float32)
    rowi = lax.broadcasted_iota(jnp.int32, (N_EXP, N_TOK), 0)
    maxv = jnp.max(lg, axis=0, keepdims=True)
    eid = jnp.min(jnp.where(lg == maxv, rowi, N_EXP), axis=0, keepdims=True)
    onehot = (rowi == eid).astype(jnp.float32)
    c = onehot
    d = 1
    while d < N_TOK:
        c = c + jnp.concatenate(
            [jnp.zeros((N_EXP, d), jnp.float32), c[:, : N_TOK - d]], axis=1)
        d *= 2
    rank = jnp.sum(c * onehot, axis=0, keepdims=True) - 1.0
    counts = c[:, N_TOK - 1 : N_TOK]
    pcnt = jnp.ceil(counts / 8.0) * 8.0
    ri = lax.broadcasted_iota(jnp.int32, (N_EXP, N_EXP), 0)
    ci = lax.broadcasted_iota(jnp.int32, (N_EXP, N_EXP), 1)
    tril = (ci < ri).astype(jnp.float32)
    pstart = lax.dot_general(
        tril, pcnt, (((1,), (0,)), ((), ())),
        preferred_element_type=jnp.float32)
    startn = jnp.sum(onehot * pstart, axis=0, keepdims=True)
    pos_ref[...] = (startn + rank).astype(jnp.int32)
    off_ref[...] = jnp.concatenate([pstart, counts], axis=1).astype(jnp.int32)


def _route(x_flat, wr):
    pos2d, offcnt = pl.pallas_call(
        _route_kernel,
        out_shape=(
            jax.ShapeDtypeStruct((1, N_TOK), jnp.int32),
            jax.ShapeDtypeStruct((N_EXP, 2), jnp.int32),
        ),
    )(x_flat, wr)
    return pos2d.reshape(N_TOK), offcnt.reshape(2 * N_EXP)


def _sc_mesh():
    info = plsc.get_sparse_core_info()
    return (
        plsc.VectorSubcoreMesh(core_axis_name="c", subcore_axis_name="s"),
        info.num_cores,
        info.num_cores * info.num_subcores,
    )


def _sc_scatter(x_flat, pos):
    mesh, nc, nw = _sc_mesh()
    bpw = N_TOK // nw

    @functools.partial(
        pl.kernel,
        out_type=jax.ShapeDtypeStruct((NPAD, N_EMB), jnp.float32),
        mesh=mesh,
        scratch_types=[
            pltpu.VMEM((bpw,), jnp.int32),
            pltpu.VMEM((bpw, N_EMB), jnp.float32),
            pltpu.SemaphoreType.DMA,
        ],
    )
    def k(x_hbm, idx_hbm, out_hbm, idx_v, rows_v, sem):
        wid = lax.axis_index("s") * nc + lax.axis_index("c")
        base = wid * bpw
        pltpu.sync_copy(idx_hbm.at[pl.ds(base, bpw)], idx_v)
        pltpu.sync_copy(x_hbm.at[pl.ds(base, bpw)], rows_v)
        pltpu.async_copy(rows_v, out_hbm.at[idx_v], sem).wait()

    return k(x_flat, pos)


def _sc_gather(table, pos):
    mesh, nc, nw = _sc_mesh()
    bpw = N_TOK // nw

    @functools.partial(
        pl.kernel,
        out_type=jax.ShapeDtypeStruct((N_TOK, N_EMB), jnp.float32),
        mesh=mesh,
        scratch_types=[
            pltpu.VMEM((bpw,), jnp.int32),
            pltpu.VMEM((bpw, N_EMB), jnp.float32),
            pltpu.SemaphoreType.DMA,
        ],
    )
    def k(table_hbm, idx_hbm, out_hbm, idx_v, rows_v, sem):
        wid = lax.axis_index("s") * nc + lax.axis_index("c")
        base = wid * bpw
        pltpu.sync_copy(idx_hbm.at[pl.ds(base, bpw)], idx_v)
        pltpu.async_copy(table_hbm.at[idx_v], rows_v, sem).wait()
        pltpu.sync_copy(rows_v, out_hbm.at[pl.ds(base, bpw)])

    return k(table, pos)


GEXP = 8


def _mm_kernel(off_ref, xs_ref, w1a_ref, w1b_ref, w2_ref, out_ref):
    g = pl.program_id(0)

    for j in range(GEXP):
        e = g * GEXP + j
        start = off_ref[2 * e]
        cnt = off_ref[2 * e + 1]
        w1a = w1a_ref[j]
        w1b = w1b_ref[j]
        w2 = w2_ref[j]
        nt = (cnt + TILE - 1) // TILE

        def body(t, carry, w1a=w1a, w1b=w1b, w2=w2, start=start):
            base = pl.multiple_of(start + t * TILE, 8)
            xt = xs_ref[pl.ds(base, TILE), :]
            h = (jnp.dot(xt[:, : N_EMB // 2], w1a,
                         preferred_element_type=jnp.float32)
                 + jnp.dot(xt[:, N_EMB // 2 :], w1b,
                           preferred_element_type=jnp.float32))
            h = jnp.maximum(h, 0.0)
            out_ref[pl.ds(base, TILE), :] = jnp.dot(
                h * h, w2, preferred_element_type=jnp.float32)
            return carry

        body(0, 0)
        lax.fori_loop(1, nt, body, 0)


def _grouped_mm(x_sorted, w1, w2, offcnt):
    grid_spec = pltpu.PrefetchScalarGridSpec(
        num_scalar_prefetch=1,
        grid=(N_EXP // GEXP,),
        in_specs=[
            pl.BlockSpec((NPAD, N_EMB), lambda e, off: (0, 0)),
            pl.BlockSpec((GEXP, N_EMB // 2, D_EXP), lambda e, off: (e, 0, 0)),
            pl.BlockSpec((GEXP, N_EMB // 2, D_EXP), lambda e, off: (e, 1, 0)),
            pl.BlockSpec((GEXP, D_EXP, N_EMB), lambda e, off: (e, 0, 0)),
        ],
        out_specs=pl.BlockSpec((NPAD, N_EMB), lambda e, off: (0, 0)),
    )
    return pl.pallas_call(
        _mm_kernel,
        grid_spec=grid_spec,
        out_shape=jax.ShapeDtypeStruct((NPAD, N_EMB), jnp.float32),
    )(offcnt, x_sorted, w1, w1, w2)


def kernel(x, Wr, W1, W2):
    b, t, c = x.shape
    x_flat = x.reshape(b * t, c)
    pos, offcnt = _route(x_flat, Wr)
    x_sorted = _sc_scatter(x_flat, pos)
    out_sorted = _grouped_mm(x_sorted, W1, W2, offcnt)
    out_flat = _sc_gather(out_sorted, pos)
    return out_flat.reshape(b, t, c)
